# baseline (speedup 1.0000x reference)
import jax
import jax.numpy as jnp
from jax.experimental import pallas as pl

N_NODES = 10000
N_LAYERS = 2


def _ident_body(x_ref, o_ref):
    o_ref[...] = x_ref[...]


def kernel(x, edge_index, edge_types, edge_weights, W_rel, W_self, b):
    src = edge_index[0]
    dst = edge_index[1]
    h = x
    for l in range(N_LAYERS):
        hr = jnp.einsum('nd,rdh->rnh', h, W_rel[l])
        msg = hr[edge_types, src] * edge_weights[:, None]
        agg = jax.ops.segment_sum(msg, dst, num_segments=N_NODES)
        deg = jax.ops.segment_sum(edge_weights, dst, num_segments=N_NODES)
        agg = agg / (deg[:, None] + 1e-6)
        h = jax.nn.relu(agg + h @ W_self[l] + b[l]) + h
    out = pl.pallas_call(
        _ident_body,
        out_shape=jax.ShapeDtypeStruct(h.shape, h.dtype),
    )(h)
    return out


# speedup vs baseline: 8.7333x; 8.7333x over previous
import jax
import jax.numpy as jnp
from jax import lax
from jax.experimental import pallas as pl
from jax.experimental.pallas import tpu as pltpu
from jax.experimental.pallas import tpu_sc as plsc

N = 10000
D = 128
R = 8
E = 320000
NLAYERS = 2

N_PAD = 10240
CH = 128
NW = 32
E_PER_W = 10240
E_PAD = NW * E_PER_W
NCHUNK = E_PER_W // CH
RPT = N_PAD // 16

_BN = 1280


def _mm_body(h_ref, w_ref, o_ref):
    o_ref[...] = jnp.dot(h_ref[...], w_ref[...],
                         preferred_element_type=jnp.float32)


def _tc_relation_table(h, w_cat):
    return pl.pallas_call(
        _mm_body,
        grid=(N_PAD // _BN,),
        in_specs=[
            pl.BlockSpec((_BN, D), lambda i: (i, 0)),
            pl.BlockSpec((D, R * D), lambda i: (0, 0)),
        ],
        out_specs=pl.BlockSpec((_BN, R * D), lambda i: (i, 0)),
        out_shape=jax.ShapeDtypeStruct((N_PAD, R * D), jnp.float32),
    )(h, w_cat)


def _fin_body(part_ref, degp_ref, h_ref, w_ref, b_ref, o_ref):
    p = part_ref[...]
    agg = p[0] + p[1]
    dg = degp_ref[...]
    deg = dg[0] + dg[1]
    h = h_ref[...]
    agg = agg / (deg[:, None] + 1e-6)
    o_ref[...] = jax.nn.relu(
        agg + jnp.dot(h, w_ref[...], preferred_element_type=jnp.float32)
        + b_ref[...]) + h


def _tc_finish(part, degp, h, w_self, b):
    return pl.pallas_call(
        _fin_body,
        grid=(N_PAD // _BN,),
        in_specs=[
            pl.BlockSpec((2, _BN, D), lambda i: (0, i, 0)),
            pl.BlockSpec((2, _BN), lambda i: (0, i)),
            pl.BlockSpec((_BN, D), lambda i: (i, 0)),
            pl.BlockSpec((D, D), lambda i: (0, 0)),
            pl.BlockSpec((1, D), lambda i: (0, 0)),
        ],
        out_specs=pl.BlockSpec((_BN, D), lambda i: (i, 0)),
        out_shape=jax.ShapeDtypeStruct((N_PAD, D), jnp.float32),
    )(part, degp, h, w_self, b)


def _sc_body(hr_hbm, gidx_hbm, dst_hbm, w_hbm, zeros_hbm, zeros1_hbm,
             out_hbm, outdeg_hbm,
             gidx_v, dst_v, w_v, rows_v, acc, accd, sem):
    cid = lax.axis_index("c")
    sid = lax.axis_index("s")
    wid = cid * 16 + sid

    zoff = sid * RPT
    pltpu.sync_copy(zeros_hbm.at[pl.ds(zoff, RPT)], acc.at[pl.ds(zoff, RPT)])
    pltpu.sync_copy(zeros1_hbm.at[pl.ds(zoff, RPT)],
                    accd.at[pl.ds(zoff, RPT)])
    plsc.subcore_barrier()

    base = wid * E_PER_W

    def chunk(c, carry):
        off = pl.multiple_of(base + c * CH, 8)
        pltpu.sync_copy(gidx_hbm.at[pl.ds(off, CH)], gidx_v)
        pltpu.sync_copy(dst_hbm.at[pl.ds(off, CH)], dst_v)
        pltpu.sync_copy(w_hbm.at[pl.ds(off, CH)], w_v)
        pltpu.async_copy(hr_hbm.at[gidx_v], rows_v, sem).wait()

        def group(g, carry2):
            wv = w_v[pl.ds(g * 16, 16)]
            for j in range(16):
                w = wv[j]
                e = g * 16 + j
                for d in range(D // 16):
                    rows_v[e, pl.ds(d * 16, 16)] = (
                        rows_v[e, pl.ds(d * 16, 16)] * w)
            return carry2

        lax.fori_loop(0, CH // 16, group, 0)
        pltpu.sync_copy(rows_v, acc.at[dst_v], add=True)
        pltpu.sync_copy(w_v, accd.at[dst_v], add=True)
        return carry

    lax.fori_loop(0, NCHUNK, chunk, 0)
    plsc.subcore_barrier()
    pltpu.sync_copy(acc.at[pl.ds(zoff, RPT)],
                    out_hbm.at[cid, pl.ds(zoff, RPT)])
    pltpu.sync_copy(accd.at[pl.ds(zoff, RPT)],
                    outdeg_hbm.at[cid, pl.ds(zoff, RPT)])


_sc_aggregate = pl.kernel(
    _sc_body,
    mesh=plsc.VectorSubcoreMesh(core_axis_name="c", subcore_axis_name="s"),
    out_type=[
        jax.ShapeDtypeStruct((2, N_PAD, D), jnp.float32),
        jax.ShapeDtypeStruct((2, N_PAD), jnp.float32),
    ],
    scratch_types=[
        pltpu.VMEM((CH,), jnp.int32),
        pltpu.VMEM((CH,), jnp.int32),
        pltpu.VMEM((CH,), jnp.float32),
        pltpu.VMEM((CH, D), jnp.float32),
        pltpu.VMEM_SHARED((N_PAD, D), jnp.float32),
        pltpu.VMEM_SHARED((N_PAD,), jnp.float32),
        pltpu.SemaphoreType.DMA,
    ],
)


def kernel(x, edge_index, edge_types, edge_weights, W_rel, W_self, b):
    src = edge_index[0].astype(jnp.int32)
    dst = edge_index[1].astype(jnp.int32)
    et = edge_types.astype(jnp.int32)
    gidx = src * R + et
    npad = E_PAD - E
    gidx = jnp.concatenate([gidx, jnp.zeros((npad,), jnp.int32)])
    dstp = jnp.concatenate([dst, jnp.zeros((npad,), jnp.int32)])
    wp = jnp.concatenate([edge_weights.astype(jnp.float32),
                          jnp.zeros((npad,), jnp.float32)])
    zeros = jnp.zeros((N_PAD, D), jnp.float32)
    zeros1 = jnp.zeros((N_PAD,), jnp.float32)

    h = jnp.concatenate([x, jnp.zeros((N_PAD - N, D), jnp.float32)])
    for l in range(NLAYERS):
        w_cat = W_rel[l].transpose(1, 0, 2).reshape(D, R * D)
        hr = _tc_relation_table(h, w_cat).reshape(N_PAD * R, D)
        part, degp = _sc_aggregate(hr, gidx, dstp, wp, zeros, zeros1)
        h = _tc_finish(part, degp, h, W_self[l], b[l].reshape(1, D))
    return h[:N]


# speedup vs baseline: 11.8548x; 1.3574x over previous
import jax
import jax.numpy as jnp
from jax import lax
from jax.experimental import pallas as pl
from jax.experimental.pallas import tpu as pltpu
from jax.experimental.pallas import tpu_sc as plsc

N = 10000
D = 128
R = 8
E = 320000
NLAYERS = 2

N_PAD = 10240
CH = 128
NW = 32
E_PER_W = 10240
E_PAD = NW * E_PER_W
NCHUNK = E_PER_W // CH
RPT = N_PAD // 16

_BN = 1280


def _mm_body(h_ref, w_ref, o_ref):
    o_ref[...] = jnp.dot(h_ref[...], w_ref[...],
                         preferred_element_type=jnp.float32)


def _tc_relation_table(h, w_cat):
    return pl.pallas_call(
        _mm_body,
        grid=(N_PAD // _BN,),
        in_specs=[
            pl.BlockSpec((_BN, D), lambda i: (i, 0)),
            pl.BlockSpec((D, R * D), lambda i: (0, 0)),
        ],
        out_specs=pl.BlockSpec((_BN, R * D), lambda i: (i, 0)),
        out_shape=jax.ShapeDtypeStruct((N_PAD, R * D), jnp.float32),
    )(h, w_cat)


def _fin_body(part_ref, degp_ref, h_ref, w_ref, b_ref, o_ref):
    p = part_ref[...]
    agg = p[0] + p[1]
    dg = degp_ref[...]
    deg = dg[0] + dg[1]
    h = h_ref[...]
    agg = agg / (deg[:, None] + 1e-6)
    o_ref[...] = jax.nn.relu(
        agg + jnp.dot(h, w_ref[...], preferred_element_type=jnp.float32)
        + b_ref[...]) + h


def _tc_finish(part, degp, h, w_self, b):
    return pl.pallas_call(
        _fin_body,
        grid=(N_PAD // _BN,),
        in_specs=[
            pl.BlockSpec((2, _BN, D), lambda i: (0, i, 0)),
            pl.BlockSpec((2, _BN), lambda i: (0, i)),
            pl.BlockSpec((_BN, D), lambda i: (i, 0)),
            pl.BlockSpec((D, D), lambda i: (0, 0)),
            pl.BlockSpec((1, D), lambda i: (0, 0)),
        ],
        out_specs=pl.BlockSpec((_BN, D), lambda i: (i, 0)),
        out_shape=jax.ShapeDtypeStruct((N_PAD, D), jnp.float32),
    )(part, degp, h, w_self, b)


NITER = NCHUNK // 2


def _make_sc(with_deg):
    def body(*refs):
        if with_deg:
            (hr_hbm, gidx_hbm, dst_hbm, w_hbm, zeros_hbm, zeros1_hbm,
             out_hbm, outdeg_hbm,
             gidx_v, dst4, w4, rows0, rows1, acc, accd,
             gsem0, gsem1, isem0, isem1, isem2, isem3) = refs
        else:
            (hr_hbm, gidx_hbm, dst_hbm, w_hbm, zeros_hbm,
             out_hbm,
             gidx_v, dst4, w4, rows0, rows1, acc,
             gsem0, gsem1, isem0, isem1, isem2, isem3) = refs
        rows = (rows0, rows1)
        gsem = (gsem0, gsem1)
        isem = (isem0, isem1, isem2, isem3)

        cid = lax.axis_index("c")
        sid = lax.axis_index("s")
        wid = cid * 16 + sid

        zoff = sid * RPT
        pltpu.sync_copy(zeros_hbm.at[pl.ds(zoff, RPT)],
                        acc.at[pl.ds(zoff, RPT)])
        if with_deg:
            pltpu.sync_copy(zeros1_hbm.at[pl.ds(zoff, RPT)],
                            accd.at[pl.ds(zoff, RPT)])
        base = pl.multiple_of(wid * E_PER_W, 8)
        pltpu.sync_copy(gidx_hbm.at[pl.ds(base, E_PER_W)], gidx_v)
        plsc.subcore_barrier()

        def gather(c, b):
            off = pl.multiple_of(c * CH, 8)
            pltpu.async_copy(hr_hbm.at[gidx_v.at[pl.ds(off, CH)]],
                             rows[b], gsem[b])

        def wait_gather(c, b):
            off = pl.multiple_of(c * CH, 8)
            pltpu.make_async_copy(hr_hbm.at[gidx_v.at[pl.ds(off, CH)]],
                                  rows[b], gsem[b]).wait()

        def load_idx(c, q):
            off = pl.multiple_of(base + c * CH, 8)
            pltpu.async_copy(w_hbm.at[pl.ds(off, CH)], w4.at[q], isem[q])
            pltpu.async_copy(dst_hbm.at[pl.ds(off, CH)], dst4.at[q],
                             isem[q])

        def wait_idx(c, q):
            off = pl.multiple_of(base + c * CH, 8)
            pltpu.make_async_copy(w_hbm.at[pl.ds(off, CH)], w4.at[q],
                                  isem[q]).wait()
            pltpu.make_async_copy(dst_hbm.at[pl.ds(off, CH)], dst4.at[q],
                                  isem[q]).wait()

        def do_chunk(c, b, q):
            wait_gather(c, b)
            wait_idx(c, q)

            def group(g, carry2):
                wv = w4[q, pl.ds(g * 16, 16)]
                for j in range(16):
                    w = wv[j]
                    e = g * 16 + j
                    for d in range(D // 16):
                        rows[b][e, pl.ds(d * 16, 16)] = (
                            rows[b][e, pl.ds(d * 16, 16)] * w)
                return carry2

            lax.fori_loop(0, CH // 16, group, 0)
            pltpu.sync_copy(rows[b], acc.at[dst4.at[q]], add=True)
            if with_deg:
                pltpu.sync_copy(w4.at[q], accd.at[dst4.at[q]], add=True)

            @pl.when(c + 2 < NCHUNK)
            def _():
                gather(c + 2, b)
                load_idx(c + 2, (q + 2) % 4)

        load_idx(0, 0)
        load_idx(1, 1)
        gather(0, 0)
        gather(1, 1)

        def quad(i, carry):
            c0 = i * 4
            for k in range(4):
                do_chunk(c0 + k, k % 2, k)
            return carry

        lax.fori_loop(0, NCHUNK // 4, quad, 0)
        plsc.subcore_barrier()
        pltpu.sync_copy(acc.at[pl.ds(zoff, RPT)],
                        out_hbm.at[cid, pl.ds(zoff, RPT)])
        if with_deg:
            pltpu.sync_copy(accd.at[pl.ds(zoff, RPT)],
                            outdeg_hbm.at[cid, pl.ds(zoff, RPT)])

    out_type = [jax.ShapeDtypeStruct((2, N_PAD, D), jnp.float32)]
    if with_deg:
        out_type.append(jax.ShapeDtypeStruct((2, N_PAD), jnp.float32))
    scratch = [
        pltpu.VMEM((E_PER_W,), jnp.int32),
        pltpu.VMEM((4, CH), jnp.int32),
        pltpu.VMEM((4, CH), jnp.float32),
        pltpu.VMEM((CH, D), jnp.float32),
        pltpu.VMEM((CH, D), jnp.float32),
        pltpu.VMEM_SHARED((N_PAD, D), jnp.float32),
    ]
    if with_deg:
        scratch.append(pltpu.VMEM_SHARED((N_PAD,), jnp.float32))
    scratch += [pltpu.SemaphoreType.DMA] * 6
    return pl.kernel(
        body,
        mesh=plsc.VectorSubcoreMesh(core_axis_name="c",
                                    subcore_axis_name="s"),
        out_type=out_type,
        scratch_types=scratch,
    )


_sc_aggregate_deg = _make_sc(True)
_sc_aggregate = _make_sc(False)


def kernel(x, edge_index, edge_types, edge_weights, W_rel, W_self, b):
    src = edge_index[0].astype(jnp.int32)
    dst = edge_index[1].astype(jnp.int32)
    et = edge_types.astype(jnp.int32)
    gidx = src * R + et
    npad = E_PAD - E
    gidx = jnp.concatenate([gidx, jnp.zeros((npad,), jnp.int32)])
    dstp = jnp.concatenate([dst, jnp.zeros((npad,), jnp.int32)])
    wp = jnp.concatenate([edge_weights.astype(jnp.float32),
                          jnp.zeros((npad,), jnp.float32)])
    zeros = jnp.zeros((N_PAD, D), jnp.float32)
    zeros1 = jnp.zeros((N_PAD,), jnp.float32)

    h = jnp.concatenate([x, jnp.zeros((N_PAD - N, D), jnp.float32)])
    degp = None
    for l in range(NLAYERS):
        w_cat = W_rel[l].transpose(1, 0, 2).reshape(D, R * D)
        hr = _tc_relation_table(h, w_cat).reshape(N_PAD * R, D)
        if l == 0:
            part, degp = _sc_aggregate_deg(hr, gidx, dstp, wp, zeros, zeros1)
        else:
            (part,) = _sc_aggregate(hr, gidx, dstp, wp, zeros)
        h = _tc_finish(part, degp, h, W_self[l], b[l].reshape(1, D))
    return h[:N]


# speedup vs baseline: 28.9311x; 2.4405x over previous
import jax
import jax.numpy as jnp
from jax import lax
from jax.experimental import pallas as pl
from jax.experimental.pallas import tpu as pltpu
from jax.experimental.pallas import tpu_sc as plsc

N = 10000
D = 128
R = 8
E = 320000
NLAYERS = 2

N_PAD = 10240
CH = 128
NW = 32
E_PER_W = 10240
E_PAD = NW * E_PER_W
NCHUNK = E_PER_W // CH
RPT = N_PAD // 16

_BN = 1280


def _mm_body(h_ref, w_ref, o_ref):
    o_ref[...] = jnp.dot(h_ref[...], w_ref[...],
                         preferred_element_type=jnp.float32)


def _tc_relation_table(h, w_cat):
    return pl.pallas_call(
        _mm_body,
        grid=(N_PAD // _BN,),
        in_specs=[
            pl.BlockSpec((_BN, D), lambda i: (i, 0)),
            pl.BlockSpec((D, R * D), lambda i: (0, 0)),
        ],
        out_specs=pl.BlockSpec((_BN, R * D), lambda i: (i, 0)),
        out_shape=jax.ShapeDtypeStruct((N_PAD, R * D), jnp.float32),
    )(h, w_cat)


def _fin_body(part_ref, degp_ref, h_ref, w_ref, b_ref, o_ref):
    p = part_ref[...]
    agg = p[0] + p[1]
    dg = degp_ref[...]
    deg = dg[0] + dg[1]
    h = h_ref[...]
    agg = agg / (deg[:, None] + 1e-6)
    o_ref[...] = jax.nn.relu(
        agg + jnp.dot(h, w_ref[...], preferred_element_type=jnp.float32)
        + b_ref[...]) + h


def _tc_finish(part, degp, h, w_self, b):
    return pl.pallas_call(
        _fin_body,
        grid=(N_PAD // _BN,),
        in_specs=[
            pl.BlockSpec((2, _BN, D), lambda i: (0, i, 0)),
            pl.BlockSpec((2, _BN), lambda i: (0, i)),
            pl.BlockSpec((_BN, D), lambda i: (i, 0)),
            pl.BlockSpec((D, D), lambda i: (0, 0)),
            pl.BlockSpec((1, D), lambda i: (0, 0)),
        ],
        out_specs=pl.BlockSpec((_BN, D), lambda i: (i, 0)),
        out_shape=jax.ShapeDtypeStruct((N_PAD, D), jnp.float32),
    )(part, degp, h, w_self, b)


NITER = NCHUNK // 2


def _make_sc(with_deg):
    def body(*refs):
        if with_deg:
            (hr_hbm, gidx_hbm, dst_hbm, w_hbm, zeros_hbm, zeros1_hbm,
             out_hbm, outdeg_hbm,
             gidx_v, dst4, w4, rows0, rows1, acc, accd,
             gsem0, gsem1, isem0, isem1, isem2, isem3) = refs
        else:
            (hr_hbm, gidx_hbm, dst_hbm, w_hbm, zeros_hbm,
             out_hbm,
             gidx_v, dst4, w4, rows0, rows1, acc,
             gsem0, gsem1, isem0, isem1, isem2, isem3) = refs
        rows = (rows0, rows1)
        gsem = (gsem0, gsem1)
        isem = (isem0, isem1, isem2, isem3)

        cid = lax.axis_index("c")
        sid = lax.axis_index("s")
        wid = cid * 16 + sid

        zoff = sid * RPT
        pltpu.sync_copy(zeros_hbm.at[pl.ds(zoff, RPT)],
                        acc.at[pl.ds(zoff, RPT)])
        if with_deg:
            pltpu.sync_copy(zeros1_hbm.at[pl.ds(zoff, RPT)],
                            accd.at[pl.ds(zoff, RPT)])
        base = pl.multiple_of(wid * E_PER_W, 8)
        pltpu.sync_copy(gidx_hbm.at[pl.ds(base, E_PER_W)], gidx_v)
        plsc.subcore_barrier()

        def gather(c, b):
            off = pl.multiple_of(c * CH, 8)
            pltpu.async_copy(hr_hbm.at[gidx_v.at[pl.ds(off, CH)]],
                             rows[b], gsem[b])

        def wait_gather(c, b):
            off = pl.multiple_of(c * CH, 8)
            pltpu.make_async_copy(hr_hbm.at[gidx_v.at[pl.ds(off, CH)]],
                                  rows[b], gsem[b]).wait()

        def load_idx(c, q):
            off = pl.multiple_of(base + c * CH, 8)
            pltpu.async_copy(w_hbm.at[pl.ds(off, CH)], w4.at[q], isem[q])
            pltpu.async_copy(dst_hbm.at[pl.ds(off, CH)], dst4.at[q],
                             isem[q])

        def wait_idx(c, q):
            off = pl.multiple_of(base + c * CH, 8)
            pltpu.make_async_copy(w_hbm.at[pl.ds(off, CH)], w4.at[q],
                                  isem[q]).wait()
            pltpu.make_async_copy(dst_hbm.at[pl.ds(off, CH)], dst4.at[q],
                                  isem[q]).wait()

        def do_chunk(c, b, q):
            wait_gather(c, b)
            wait_idx(c, q)

            def group(g, carry2):
                wv = w4[q, pl.ds(g * 16, 16)]
                for j in range(16):
                    w = wv[j]
                    e = g * 16 + j
                    for d in range(D // 16):
                        rows[b][e, pl.ds(d * 16, 16)] = (
                            rows[b][e, pl.ds(d * 16, 16)] * w)
                return carry2

            lax.fori_loop(0, CH // 16, group, 0)
            pltpu.sync_copy(rows[b], acc.at[dst4.at[q]], add=True)
            if with_deg:
                pltpu.sync_copy(w4.at[q], accd.at[dst4.at[q]], add=True)

            @pl.when(c + 2 < NCHUNK)
            def _():
                gather(c + 2, b)
                load_idx(c + 2, (q + 2) % 4)

        load_idx(0, 0)
        load_idx(1, 1)
        gather(0, 0)
        gather(1, 1)

        def quad(i, carry):
            c0 = i * 4
            for k in range(4):
                do_chunk(c0 + k, k % 2, k)
            return carry

        lax.fori_loop(0, NCHUNK // 4, quad, 0)
        plsc.subcore_barrier()
        pltpu.sync_copy(acc.at[pl.ds(zoff, RPT)],
                        out_hbm.at[cid, pl.ds(zoff, RPT)])
        if with_deg:
            pltpu.sync_copy(accd.at[pl.ds(zoff, RPT)],
                            outdeg_hbm.at[cid, pl.ds(zoff, RPT)])

    out_type = [jax.ShapeDtypeStruct((2, N_PAD, D), jnp.float32)]
    if with_deg:
        out_type.append(jax.ShapeDtypeStruct((2, N_PAD), jnp.float32))
    scratch = [
        pltpu.VMEM((E_PER_W,), jnp.int32),
        pltpu.VMEM((4, CH), jnp.int32),
        pltpu.VMEM((4, CH), jnp.float32),
        pltpu.VMEM((CH, D), jnp.float32),
        pltpu.VMEM((CH, D), jnp.float32),
        pltpu.VMEM_SHARED((N_PAD, D), jnp.float32),
    ]
    if with_deg:
        scratch.append(pltpu.VMEM_SHARED((N_PAD,), jnp.float32))
    scratch += [pltpu.SemaphoreType.DMA] * 6
    return pl.kernel(
        body,
        mesh=plsc.VectorSubcoreMesh(core_axis_name="c",
                                    subcore_axis_name="s"),
        out_type=out_type,
        scratch_types=scratch,
    )


_sc_aggregate_deg = _make_sc(True)
_sc_aggregate = _make_sc(False)


def kernel(x, edge_index, edge_types, edge_weights, W_rel, W_self, b):
    src = edge_index[0].astype(jnp.int32)
    dst = edge_index[1].astype(jnp.int32)
    et = edge_types.astype(jnp.int32)
    gidx = src * R + et
    npad = E_PAD - E
    spread = jnp.arange(npad, dtype=jnp.int32) % N
    gidx = jnp.concatenate([gidx, spread * R])
    dstp = jnp.concatenate([dst, spread])
    wp = jnp.concatenate([edge_weights.astype(jnp.float32),
                          jnp.zeros((npad,), jnp.float32)])
    zeros = jnp.zeros((N_PAD, D), jnp.float32)
    zeros1 = jnp.zeros((N_PAD,), jnp.float32)

    h = jnp.concatenate([x, jnp.zeros((N_PAD - N, D), jnp.float32)])
    degp = None
    for l in range(NLAYERS):
        w_cat = W_rel[l].transpose(1, 0, 2).reshape(D, R * D)
        hr = _tc_relation_table(h, w_cat).reshape(N_PAD * R, D)
        if l == 0:
            part, degp = _sc_aggregate_deg(hr, gidx, dstp, wp, zeros, zeros1)
        else:
            (part,) = _sc_aggregate(hr, gidx, dstp, wp, zeros)
        h = _tc_finish(part, degp, h, W_self[l], b[l].reshape(1, D))
    return h[:N]
